# Initial kernel scaffold; baseline (speedup 1.0000x reference)
import functools

import jax
import jax.numpy as jnp
from jax import lax
from jax.experimental import pallas as pl
from jax.experimental.pallas import tpu as pltpu
from jax.experimental.pallas import tpu_sc as plsc

_MIN_NORM = 1e-15
_MAXNORM = 1.0 - 4e-3
_ATANH_CLIP = 1.0 - 1e-7

_NC = 2
_NS = 16
_NW = _NC * _NS
_L = 16
_C = 128


def _rn(x):
    return jnp.maximum(jnp.sqrt(jnp.sum(x * x, axis=-1, keepdims=True)),
                       _MIN_NORM)


def _atanh(x):
    return 0.5 * jnp.log((1.0 + x) / (1.0 - x))


def _proj(x):
    n = _rn(x)
    return jnp.where(n > _MAXNORM, x * (_MAXNORM / n), x)


def _expmap0(u):
    n = _rn(u)
    return jnp.tanh(n) * u / n


def _logmap0(p):
    n = _rn(p)
    return _atanh(jnp.minimum(n, _ATANH_CLIP)) * p / n


def _hyp_linear(x, W, b):
    xn = _rn(x)
    mx = lax.dot_general(x, W, (((1,), (1,)), ((), ())),
                         precision=lax.Precision.HIGHEST)
    mxn = _rn(mx)
    res = jnp.tanh(mxn / xn * _atanh(jnp.minimum(xn, _ATANH_CLIP))) * mx / mxn
    zero_rows = jnp.all(mx == 0.0, axis=-1, keepdims=True)
    mv = _proj(jnp.where(zero_rows, 0.0, res))
    hyp_b = _proj(_expmap0(b))
    x2 = jnp.sum(mv * mv, axis=-1, keepdims=True)
    y2 = jnp.sum(hyp_b * hyp_b, axis=-1, keepdims=True)
    xy = jnp.sum(mv * hyp_b, axis=-1, keepdims=True)
    num = (1.0 + 2.0 * xy + y2) * mv + (1.0 - x2) * hyp_b
    den = 1.0 + 2.0 * xy + x2 * y2
    return _proj(num / jnp.maximum(den, _MIN_NORM))


def _segment_mean_in(p0, p1, d0, d1):
    deg = jnp.maximum(d0[:, :1] + d1[:, :1], 1.0)
    return (p0 + p1) / deg


def _stage1_body(x_ref, w_ref, b_ref, o_ref):
    xh = _proj(_expmap0(x_ref[...]))
    h = _hyp_linear(xh, w_ref[...], b_ref[...])
    o_ref[...] = _logmap0(h)


def _stage2_body(p0_ref, p1_ref, d0_ref, d1_ref, w_ref, b_ref, o_ref):
    support = _segment_mean_in(p0_ref[...], p1_ref[...],
                               d0_ref[...], d1_ref[...])
    h = _proj(_expmap0(support))
    xt = jnp.maximum(_logmap0(h), 0.0)
    h2 = _proj(_expmap0(xt))
    h3 = _hyp_linear(h2, w_ref[...], b_ref[...])
    o_ref[...] = _logmap0(h3)


def _stage3_body(p0_ref, p1_ref, d0_ref, d1_ref, o_ref):
    support = _segment_mean_in(p0_ref[...], p1_ref[...],
                               d0_ref[...], d1_ref[...])
    h = _proj(_expmap0(support))
    xt = jnp.maximum(_logmap0(h), 0.0)
    o_ref[...] = _proj(_expmap0(xt))


_TC_INTERPRET = False


def _tc_stage1(x, W, b, block_rows=500):
    n, d = x.shape
    return pl.pallas_call(
        _stage1_body,
        grid=(n // block_rows,),
        in_specs=[pl.BlockSpec((block_rows, d), lambda i: (i, 0)),
                  pl.BlockSpec((d, d), lambda i: (0, 0)),
                  pl.BlockSpec((1, d), lambda i: (0, 0))],
        out_specs=pl.BlockSpec((block_rows, d), lambda i: (i, 0)),
        out_shape=jax.ShapeDtypeStruct((n, d), jnp.float32),
        interpret=_TC_INTERPRET,
    )(x, W, b[None, :])


def _tc_stage2(p0, p1, d0, d1, W, b, block_rows=500):
    n, d = p0.shape
    return pl.pallas_call(
        _stage2_body,
        grid=(n // block_rows,),
        in_specs=[pl.BlockSpec((block_rows, d), lambda i: (i, 0)),
                  pl.BlockSpec((block_rows, d), lambda i: (i, 0)),
                  pl.BlockSpec((block_rows, _L), lambda i: (i, 0)),
                  pl.BlockSpec((block_rows, _L), lambda i: (i, 0)),
                  pl.BlockSpec((d, d), lambda i: (0, 0)),
                  pl.BlockSpec((1, d), lambda i: (0, 0))],
        out_specs=pl.BlockSpec((block_rows, d), lambda i: (i, 0)),
        out_shape=jax.ShapeDtypeStruct((n, d), jnp.float32),
        interpret=_TC_INTERPRET,
    )(p0, p1, d0, d1, W, b[None, :])


def _tc_stage3(p0, p1, d0, d1, block_rows=500):
    n, d = p0.shape
    return pl.pallas_call(
        _stage3_body,
        grid=(n // block_rows,),
        in_specs=[pl.BlockSpec((block_rows, d), lambda i: (i, 0)),
                  pl.BlockSpec((block_rows, d), lambda i: (i, 0)),
                  pl.BlockSpec((block_rows, _L), lambda i: (i, 0)),
                  pl.BlockSpec((block_rows, _L), lambda i: (i, 0))],
        out_specs=pl.BlockSpec((block_rows, d), lambda i: (i, 0)),
        out_shape=jax.ShapeDtypeStruct((n, d), jnp.float32),
        interpret=_TC_INTERPRET,
    )(p0, p1, d0, d1)


def _sc_agg(x_t, src3, dst3, n_nodes):
    d = x_t.shape[1]
    n_chunks = src3.shape[1]
    np1 = n_nodes + 1
    rows_per_tile = n_nodes // _NS
    zr = 125
    assert rows_per_tile % zr == 0

    mesh = plsc.VectorSubcoreMesh(core_axis_name="c", subcore_axis_name="s")

    @functools.partial(
        pl.kernel,
        out_type=(jax.ShapeDtypeStruct((_NC * n_nodes, d), jnp.float32),
                  jax.ShapeDtypeStruct((_NC * n_nodes, _L), jnp.float32)),
        mesh=mesh,
        scratch_types=[
            pltpu.VMEM((n_chunks, _C), jnp.int32),
            pltpu.VMEM((n_chunks, _C), jnp.int32),
            pltpu.VMEM((_C, d), jnp.float32),
            pltpu.VMEM((_C, _L), jnp.float32),
            pltpu.VMEM((zr, d), jnp.float32),
            pltpu.VMEM((zr, _L), jnp.float32),
            pltpu.VMEM_SHARED((np1, d), jnp.float32),
            pltpu.VMEM_SHARED((np1, _L), jnp.float32),
            pltpu.SemaphoreType.DMA,
        ],
    )
    def k(xt_hbm, src_hbm, dst_hbm, out_hbm, deg_hbm,
          src_v, dst_v, rows_v, ones_v, zb_v, zd_v, acc_sh, deg_sh, sem):
        c = lax.axis_index("c")
        s = lax.axis_index("s")
        wid = c * _NS + s

        zero16 = jnp.zeros((_L,), jnp.float32)
        one16 = jnp.ones((_L,), jnp.float32)

        def _zrow(i, _):
            for q in range(d // _L):
                zb_v[i, pl.ds(q * _L, _L)] = zero16
            zd_v[i, :] = zero16
            return 0
        lax.fori_loop(0, zr, _zrow, 0)

        def _orow(i, _):
            ones_v[i, :] = one16
            return 0
        lax.fori_loop(0, _C, _orow, 0)

        row0 = s * rows_per_tile
        for t in range(rows_per_tile // zr):
            pltpu.sync_copy(zb_v, acc_sh.at[pl.ds(row0 + t * zr, zr)])
            pltpu.sync_copy(zd_v, deg_sh.at[pl.ds(row0 + t * zr, zr)])
        plsc.subcore_barrier()

        pltpu.sync_copy(src_hbm.at[wid], src_v)
        pltpu.sync_copy(dst_hbm.at[wid], dst_v)

        def _edge(j, _):
            pltpu.async_copy(xt_hbm.at[src_v.at[j]], rows_v, sem).wait()
            pltpu.sync_copy(rows_v, acc_sh.at[dst_v.at[j]], add=True)
            pltpu.sync_copy(ones_v, deg_sh.at[dst_v.at[j]], add=True)
            return 0
        lax.fori_loop(0, n_chunks, _edge, 0)
        plsc.subcore_barrier()

        base = c * n_nodes + row0
        pltpu.sync_copy(acc_sh.at[pl.ds(row0, rows_per_tile)],
                        out_hbm.at[pl.ds(base, rows_per_tile)])
        pltpu.sync_copy(deg_sh.at[pl.ds(row0, rows_per_tile)],
                        deg_hbm.at[pl.ds(base, rows_per_tile)])

    acc, deg = k(x_t, src3, dst3)
    acc = acc.reshape(_NC, n_nodes, d)
    deg = deg.reshape(_NC, n_nodes, _L)
    return acc, deg


def kernel(x, edge_index, W1, b1, W2, b2):
    n, d = x.shape
    e = edge_index.shape[1]
    per_tile = -(-e // (_NW * _C)) * _C
    n_chunks = per_tile // _C
    pad = _NW * per_tile - e

    ei = edge_index.astype(jnp.int32)
    src = jnp.concatenate([ei[0], jnp.zeros((pad,), jnp.int32)])
    dst = jnp.concatenate([ei[1], jnp.full((pad,), n, jnp.int32)])
    src3 = src.reshape(_NW, n_chunks, _C)
    dst3 = dst.reshape(_NW, n_chunks, _C)

    xt1 = _tc_stage1(x, W1, b1)
    acc, deg = _sc_agg(xt1, src3, dst3, n)
    xt2 = _tc_stage2(acc[0], acc[1], deg[0], deg[1], W2, b2)
    acc2, deg2 = _sc_agg(xt2, src3, dst3, n)
    return _tc_stage3(acc2[0], acc2[1], deg2[0], deg2[1])


# speedup vs baseline: 5.5648x; 5.5648x over previous
import functools

import jax
import jax.numpy as jnp
from jax import lax
from jax.experimental import pallas as pl
from jax.experimental.pallas import tpu as pltpu
from jax.experimental.pallas import tpu_sc as plsc

_MIN_NORM = 1e-15
_MAXNORM = 1.0 - 4e-3
_ATANH_CLIP = 1.0 - 1e-7

_NC = 2
_NS = 16
_NW = _NC * _NS
_L = 16
_C = 128


def _rn(x):
    return jnp.maximum(jnp.sqrt(jnp.sum(x * x, axis=-1, keepdims=True)),
                       _MIN_NORM)


def _atanh(x):
    return 0.5 * jnp.log((1.0 + x) / (1.0 - x))


def _proj(x):
    n = _rn(x)
    return jnp.where(n > _MAXNORM, x * (_MAXNORM / n), x)


def _expmap0(u):
    n = _rn(u)
    return jnp.tanh(n) * u / n


def _logmap0(p):
    n = _rn(p)
    return _atanh(jnp.minimum(n, _ATANH_CLIP)) * p / n


def _hyp_linear(x, W, b):
    xn = _rn(x)
    mx = lax.dot_general(x, W, (((1,), (1,)), ((), ())),
                         precision=lax.Precision.HIGHEST)
    mxn = _rn(mx)
    res = jnp.tanh(mxn / xn * _atanh(jnp.minimum(xn, _ATANH_CLIP))) * mx / mxn
    zero_rows = jnp.all(mx == 0.0, axis=-1, keepdims=True)
    mv = _proj(jnp.where(zero_rows, 0.0, res))
    hyp_b = _proj(_expmap0(b))
    x2 = jnp.sum(mv * mv, axis=-1, keepdims=True)
    y2 = jnp.sum(hyp_b * hyp_b, axis=-1, keepdims=True)
    xy = jnp.sum(mv * hyp_b, axis=-1, keepdims=True)
    num = (1.0 + 2.0 * xy + y2) * mv + (1.0 - x2) * hyp_b
    den = 1.0 + 2.0 * xy + x2 * y2
    return _proj(num / jnp.maximum(den, _MIN_NORM))


def _segment_mean_in(pl_, pr_, dg_):
    deg = jnp.maximum(dg_[:, :1], 1.0)
    return jnp.concatenate([pl_, pr_], axis=-1) / deg


def _stage1_body(x_ref, w_ref, b_ref, o_ref):
    xh = _proj(_expmap0(x_ref[...]))
    h = _hyp_linear(xh, w_ref[...], b_ref[...])
    o_ref[...] = _logmap0(h)


def _stage2_body(p0_ref, p1_ref, dg_ref, w_ref, b_ref, o_ref):
    support = _segment_mean_in(p0_ref[...], p1_ref[...], dg_ref[...])
    h = _proj(_expmap0(support))
    xt = jnp.maximum(_logmap0(h), 0.0)
    h2 = _proj(_expmap0(xt))
    h3 = _hyp_linear(h2, w_ref[...], b_ref[...])
    o_ref[...] = _logmap0(h3)


def _stage3_body(p0_ref, p1_ref, dg_ref, o_ref):
    support = _segment_mean_in(p0_ref[...], p1_ref[...], dg_ref[...])
    h = _proj(_expmap0(support))
    xt = jnp.maximum(_logmap0(h), 0.0)
    o_ref[...] = _proj(_expmap0(xt))


_TC_INTERPRET = False


def _tc_stage1(x, W, b, block_rows=1000):
    n, d = x.shape
    return pl.pallas_call(
        _stage1_body,
        grid=(n // block_rows,),
        in_specs=[pl.BlockSpec((block_rows, d), lambda i: (i, 0)),
                  pl.BlockSpec((d, d), lambda i: (0, 0)),
                  pl.BlockSpec((1, d), lambda i: (0, 0))],
        out_specs=pl.BlockSpec((block_rows, d), lambda i: (i, 0)),
        out_shape=jax.ShapeDtypeStruct((n, d), jnp.float32),
        interpret=_TC_INTERPRET,
    )(x, W, b[None, :])


def _tc_stage2(p0, p1, dg, W, b, block_rows=1000):
    n, dh = p0.shape
    d = 2 * dh
    return pl.pallas_call(
        _stage2_body,
        grid=(n // block_rows,),
        in_specs=[pl.BlockSpec((block_rows, dh), lambda i: (i, 0)),
                  pl.BlockSpec((block_rows, dh), lambda i: (i, 0)),
                  pl.BlockSpec((block_rows, _L), lambda i: (i, 0)),
                  pl.BlockSpec((d, d), lambda i: (0, 0)),
                  pl.BlockSpec((1, d), lambda i: (0, 0))],
        out_specs=pl.BlockSpec((block_rows, d), lambda i: (i, 0)),
        out_shape=jax.ShapeDtypeStruct((n, d), jnp.float32),
        interpret=_TC_INTERPRET,
    )(p0, p1, dg, W, b[None, :])


def _tc_stage3(p0, p1, dg, block_rows=1000):
    n, dh = p0.shape
    d = 2 * dh
    return pl.pallas_call(
        _stage3_body,
        grid=(n // block_rows,),
        in_specs=[pl.BlockSpec((block_rows, dh), lambda i: (i, 0)),
                  pl.BlockSpec((block_rows, dh), lambda i: (i, 0)),
                  pl.BlockSpec((block_rows, _L), lambda i: (i, 0))],
        out_specs=pl.BlockSpec((block_rows, d), lambda i: (i, 0)),
        out_shape=jax.ShapeDtypeStruct((n, d), jnp.float32),
        interpret=_TC_INTERPRET,
    )(p0, p1, dg)


def _sc_agg(xta, xtb, src3, dst3, n_nodes):
    dh = xta.shape[1]
    n_chunks = src3.shape[1]
    np1 = -(-(n_nodes + 1) // (_NS * 128)) * (_NS * 128)
    rows_per_tile = np1 // _NS
    zr = 128
    assert rows_per_tile % zr == 0

    mesh = plsc.VectorSubcoreMesh(core_axis_name="c", subcore_axis_name="s")

    @functools.partial(
        pl.kernel,
        out_type=(jax.ShapeDtypeStruct((_NC * np1, dh), jnp.float32),
                  jax.ShapeDtypeStruct((np1, _L), jnp.float32)),
        mesh=mesh,
        scratch_types=[
            pltpu.VMEM((n_chunks, _C), jnp.int32),
            pltpu.VMEM((n_chunks, _C), jnp.int32),
            pltpu.VMEM((_C, dh), jnp.float32),
            pltpu.VMEM((_C, _L), jnp.float32),
            pltpu.VMEM((zr, dh), jnp.float32),
            pltpu.VMEM((zr, _L), jnp.float32),
            pltpu.VMEM_SHARED((np1, dh), jnp.float32),
            pltpu.VMEM_SHARED((np1, _L), jnp.float32),
            pltpu.SemaphoreType.DMA,
        ],
        compiler_params=pltpu.CompilerParams(use_tc_tiling_on_sc=False),
    )
    def k(xta_hbm, xtb_hbm, src_hbm, dst_hbm, out_hbm, deg_hbm,
          src_v, dst_v, rows_v, ones_v, zb_v, zd_v, acc_sh, deg_sh, sem):
        c = lax.axis_index("c")
        s = lax.axis_index("s")

        zero16 = jnp.zeros((_L,), jnp.float32)
        one16 = jnp.ones((_L,), jnp.float32)

        def _zrow(i, _):
            for q in range(dh // _L):
                zb_v[i, pl.ds(q * _L, _L)] = zero16
            zd_v[i, :] = zero16
            return 0
        lax.fori_loop(0, zr, _zrow, 0)

        def _orow(i, _):
            ones_v[i, :] = one16
            return 0
        lax.fori_loop(0, _C, _orow, 0)

        row0 = s * rows_per_tile
        for t in range(rows_per_tile // zr):
            pltpu.sync_copy(zb_v, acc_sh.at[pl.ds(row0 + t * zr, zr)])
            pltpu.sync_copy(zd_v, deg_sh.at[pl.ds(row0 + t * zr, zr)])
        plsc.subcore_barrier()

        pltpu.sync_copy(src_hbm.at[s], src_v)
        pltpu.sync_copy(dst_hbm.at[s], dst_v)

        def _edge_loop(x_hbm, with_deg):
            def _edge(j, _):
                pltpu.async_copy(x_hbm.at[src_v.at[j]], rows_v, sem).wait()
                pltpu.sync_copy(rows_v, acc_sh.at[dst_v.at[j]], add=True)
                if with_deg:
                    pltpu.sync_copy(ones_v, deg_sh.at[dst_v.at[j]], add=True)
                return 0
            lax.fori_loop(0, n_chunks, _edge, 0)

        @pl.when(c == 0)
        def _():
            _edge_loop(xta_hbm, True)

        @pl.when(c == 1)
        def _():
            _edge_loop(xtb_hbm, False)

        plsc.subcore_barrier()

        pltpu.sync_copy(acc_sh.at[pl.ds(row0, rows_per_tile)],
                        out_hbm.at[pl.ds(c * np1 + row0, rows_per_tile)])

        @pl.when(c == 0)
        def _():
            pltpu.sync_copy(deg_sh.at[pl.ds(row0, rows_per_tile)],
                            deg_hbm.at[pl.ds(row0, rows_per_tile)])

    acc, deg = k(xta, xtb, src3, dst3)
    acc = acc.reshape(_NC, np1, dh)[:, :n_nodes]
    return acc, deg[:n_nodes]


def kernel(x, edge_index, W1, b1, W2, b2):
    n, d = x.shape
    dh = d // 2
    e = edge_index.shape[1]
    per_tile = -(-e // (_NS * _C)) * _C
    n_chunks = per_tile // _C
    pad = _NS * per_tile - e

    ei = edge_index.astype(jnp.int32)
    src = jnp.concatenate([ei[0], jnp.zeros((pad,), jnp.int32)])
    dst = jnp.concatenate([ei[1], jnp.full((pad,), n, jnp.int32)])
    src3 = src.reshape(_NS, n_chunks, _C)
    dst3 = dst.reshape(_NS, n_chunks, _C)

    xt1 = _tc_stage1(x, W1, b1)
    acc, deg = _sc_agg(xt1[:, :dh], xt1[:, dh:], src3, dst3, n)
    xt2 = _tc_stage2(acc[0], acc[1], deg, W2, b2)
    acc2, _ = _sc_agg(xt2[:, :dh], xt2[:, dh:], src3, dst3, n)
    return _tc_stage3(acc2[0], acc2[1], deg)


# speedup vs baseline: 6.6400x; 1.1932x over previous
import functools

import jax
import jax.numpy as jnp
from jax import lax
from jax.experimental import pallas as pl
from jax.experimental.pallas import tpu as pltpu
from jax.experimental.pallas import tpu_sc as plsc

_MIN_NORM = 1e-15
_MAXNORM = 1.0 - 4e-3
_ATANH_CLIP = 1.0 - 1e-7

_NC = 2
_NS = 16
_NW = _NC * _NS
_L = 16
_C = 128


def _rn(x):
    return jnp.maximum(jnp.sqrt(jnp.sum(x * x, axis=-1, keepdims=True)),
                       _MIN_NORM)


def _atanh(x):
    return 0.5 * jnp.log((1.0 + x) / (1.0 - x))


def _proj(x):
    n = _rn(x)
    return jnp.where(n > _MAXNORM, x * (_MAXNORM / n), x)


def _expmap0(u):
    n = _rn(u)
    return jnp.tanh(n) * u / n


def _logmap0(p):
    n = _rn(p)
    return _atanh(jnp.minimum(n, _ATANH_CLIP)) * p / n


def _hyp_linear(x, W, b):
    xn = _rn(x)
    mx = lax.dot_general(x, W, (((1,), (1,)), ((), ())),
                         precision=lax.Precision.HIGHEST)
    mxn = _rn(mx)
    res = jnp.tanh(mxn / xn * _atanh(jnp.minimum(xn, _ATANH_CLIP))) * mx / mxn
    zero_rows = jnp.all(mx == 0.0, axis=-1, keepdims=True)
    mv = _proj(jnp.where(zero_rows, 0.0, res))
    hyp_b = _proj(_expmap0(b))
    x2 = jnp.sum(mv * mv, axis=-1, keepdims=True)
    y2 = jnp.sum(hyp_b * hyp_b, axis=-1, keepdims=True)
    xy = jnp.sum(mv * hyp_b, axis=-1, keepdims=True)
    num = (1.0 + 2.0 * xy + y2) * mv + (1.0 - x2) * hyp_b
    den = 1.0 + 2.0 * xy + x2 * y2
    return _proj(num / jnp.maximum(den, _MIN_NORM))


def _segment_mean_in(pl_, pr_, dg0_, dg1_):
    deg = jnp.maximum(dg0_[:, :1] + dg1_[:, :1], 1.0)
    return jnp.concatenate([pl_, pr_], axis=-1) / deg


def _stage1_body(x_ref, w_ref, b_ref, o_ref):
    xh = _proj(_expmap0(x_ref[...]))
    h = _hyp_linear(xh, w_ref[...], b_ref[...])
    o_ref[...] = _logmap0(h)


def _stage2_body(p0_ref, p1_ref, dg0_ref, dg1_ref, w_ref, b_ref, o_ref):
    support = _segment_mean_in(p0_ref[...], p1_ref[...],
                               dg0_ref[...], dg1_ref[...])
    h = _proj(_expmap0(support))
    xt = jnp.maximum(_logmap0(h), 0.0)
    h2 = _proj(_expmap0(xt))
    h3 = _hyp_linear(h2, w_ref[...], b_ref[...])
    o_ref[...] = _logmap0(h3)


def _stage3_body(p0_ref, p1_ref, dg0_ref, dg1_ref, o_ref):
    support = _segment_mean_in(p0_ref[...], p1_ref[...],
                               dg0_ref[...], dg1_ref[...])
    h = _proj(_expmap0(support))
    xt = jnp.maximum(_logmap0(h), 0.0)
    o_ref[...] = _proj(_expmap0(xt))


_TC_INTERPRET = False


def _tc_stage1(x, W, b, block_rows=1000):
    n, d = x.shape
    return pl.pallas_call(
        _stage1_body,
        grid=(n // block_rows,),
        in_specs=[pl.BlockSpec((block_rows, d), lambda i: (i, 0)),
                  pl.BlockSpec((d, d), lambda i: (0, 0)),
                  pl.BlockSpec((1, d), lambda i: (0, 0))],
        out_specs=pl.BlockSpec((block_rows, d), lambda i: (i, 0)),
        out_shape=jax.ShapeDtypeStruct((n, d), jnp.float32),
        interpret=_TC_INTERPRET,
    )(x, W, b[None, :])


def _tc_stage2(p0, p1, dg0, dg1, W, b, block_rows=1000):
    n, dh = p0.shape
    d = 2 * dh
    return pl.pallas_call(
        _stage2_body,
        grid=(n // block_rows,),
        in_specs=[pl.BlockSpec((block_rows, dh), lambda i: (i, 0)),
                  pl.BlockSpec((block_rows, dh), lambda i: (i, 0)),
                  pl.BlockSpec((block_rows, _L), lambda i: (i, 0)),
                  pl.BlockSpec((block_rows, _L), lambda i: (i, 0)),
                  pl.BlockSpec((d, d), lambda i: (0, 0)),
                  pl.BlockSpec((1, d), lambda i: (0, 0))],
        out_specs=pl.BlockSpec((block_rows, d), lambda i: (i, 0)),
        out_shape=jax.ShapeDtypeStruct((n, d), jnp.float32),
        interpret=_TC_INTERPRET,
    )(p0, p1, dg0, dg1, W, b[None, :])


def _tc_stage3(p0, p1, dg0, dg1, block_rows=1000):
    n, dh = p0.shape
    d = 2 * dh
    return pl.pallas_call(
        _stage3_body,
        grid=(n // block_rows,),
        in_specs=[pl.BlockSpec((block_rows, dh), lambda i: (i, 0)),
                  pl.BlockSpec((block_rows, dh), lambda i: (i, 0)),
                  pl.BlockSpec((block_rows, _L), lambda i: (i, 0)),
                  pl.BlockSpec((block_rows, _L), lambda i: (i, 0))],
        out_specs=pl.BlockSpec((block_rows, d), lambda i: (i, 0)),
        out_shape=jax.ShapeDtypeStruct((n, d), jnp.float32),
        interpret=_TC_INTERPRET,
    )(p0, p1, dg0, dg1)


def _sc_agg(xta, xtb, src3, dst3, n_nodes, do_deg=True):
    dh = xta.shape[1]
    n_chunks = src3.shape[1]
    np1 = -(-(n_nodes + 1) // (_NS * 128)) * (_NS * 128)
    rows_per_tile = np1 // _NS
    zr = 128
    assert rows_per_tile % zr == 0

    mesh = plsc.VectorSubcoreMesh(core_axis_name="c", subcore_axis_name="s")
    assert n_chunks % 2 == 0

    @functools.partial(
        pl.kernel,
        out_type=(jax.ShapeDtypeStruct((_NC * np1, dh), jnp.float32),
                  jax.ShapeDtypeStruct((_NC * np1, _L), jnp.float32)),
        mesh=mesh,
        scratch_types=[
            pltpu.VMEM((n_chunks, _C), jnp.int32),
            pltpu.VMEM((n_chunks, _C), jnp.int32),
            pltpu.VMEM((_C, dh), jnp.float32),
            pltpu.VMEM((_C, dh), jnp.float32),
            pltpu.VMEM((_C, _L), jnp.float32),
            pltpu.VMEM((zr, dh), jnp.float32),
            pltpu.VMEM((zr, _L), jnp.float32),
            pltpu.VMEM_SHARED((np1, dh), jnp.float32),
            pltpu.VMEM_SHARED((np1, _L), jnp.float32),
            pltpu.SemaphoreType.DMA,
            pltpu.SemaphoreType.DMA,
        ],
        compiler_params=pltpu.CompilerParams(use_tc_tiling_on_sc=False),
    )
    def k(xta_hbm, xtb_hbm, src_hbm, dst_hbm, out_hbm, deg_hbm,
          src_v, dst_v, rows_a, rows_b, ones_v, zb_v, zd_v,
          acc_sh, deg_sh, sem_a, sem_b):
        c = lax.axis_index("c")
        s = lax.axis_index("s")

        zero16 = jnp.zeros((_L,), jnp.float32)
        one16 = jnp.ones((_L,), jnp.float32)

        def _zrow(i, _):
            for q in range(dh // _L):
                zb_v[i, pl.ds(q * _L, _L)] = zero16
            zd_v[i, :] = zero16
            return 0
        lax.fori_loop(0, zr, _zrow, 0)

        def _orow(i, _):
            ones_v[i, :] = one16
            return 0
        lax.fori_loop(0, _C, _orow, 0)

        row0 = s * rows_per_tile
        for t in range(rows_per_tile // zr):
            pltpu.sync_copy(zb_v, acc_sh.at[pl.ds(row0 + t * zr, zr)])
            pltpu.sync_copy(zd_v, deg_sh.at[pl.ds(row0 + t * zr, zr)])
        plsc.subcore_barrier()

        pltpu.sync_copy(src_hbm.at[s], src_v)
        pltpu.sync_copy(dst_hbm.at[s], dst_v)

        def _edge_loop(x_hbm, deg_on_even):
            pltpu.async_copy(x_hbm.at[src_v.at[0]], rows_a, sem_a)

            def _pair(i, _):
                ja = 2 * i
                jb = 2 * i + 1
                pltpu.async_copy(x_hbm.at[src_v.at[jb]], rows_b, sem_b)
                pltpu.make_async_copy(x_hbm.at[src_v.at[ja]],
                                      rows_a, sem_a).wait()
                pltpu.sync_copy(rows_a, acc_sh.at[dst_v.at[ja]], add=True)
                if deg_on_even is True:
                    pltpu.sync_copy(ones_v, deg_sh.at[dst_v.at[ja]],
                                    add=True)

                @pl.when(jb + 1 < n_chunks)
                def _():
                    pltpu.async_copy(x_hbm.at[src_v.at[jb + 1]],
                                     rows_a, sem_a)

                pltpu.make_async_copy(x_hbm.at[src_v.at[jb]],
                                      rows_b, sem_b).wait()
                pltpu.sync_copy(rows_b, acc_sh.at[dst_v.at[jb]], add=True)
                if deg_on_even is False:
                    pltpu.sync_copy(ones_v, deg_sh.at[dst_v.at[jb]],
                                    add=True)
                return 0
            lax.fori_loop(0, n_chunks // 2, _pair, 0)

        @pl.when(c == 0)
        def _():
            _edge_loop(xta_hbm, True if do_deg else None)

        @pl.when(c == 1)
        def _():
            _edge_loop(xtb_hbm, False if do_deg else None)

        plsc.subcore_barrier()

        pltpu.sync_copy(acc_sh.at[pl.ds(row0, rows_per_tile)],
                        out_hbm.at[pl.ds(c * np1 + row0, rows_per_tile)])
        if do_deg:
            pltpu.sync_copy(deg_sh.at[pl.ds(row0, rows_per_tile)],
                            deg_hbm.at[pl.ds(c * np1 + row0, rows_per_tile)])

    acc, deg = k(xta, xtb, src3, dst3)
    acc = acc.reshape(_NC, np1, dh)[:, :n_nodes]
    deg = deg.reshape(_NC, np1, _L)[:, :n_nodes]
    return acc, deg


def kernel(x, edge_index, W1, b1, W2, b2):
    n, d = x.shape
    dh = d // 2
    e = edge_index.shape[1]
    per_tile = -(-e // (_NS * 2 * _C)) * 2 * _C
    n_chunks = per_tile // _C
    pad = _NS * per_tile - e

    ei = edge_index.astype(jnp.int32)
    src = jnp.concatenate([ei[0], jnp.zeros((pad,), jnp.int32)])
    dst = jnp.concatenate([ei[1], jnp.full((pad,), n, jnp.int32)])
    src3 = src.reshape(_NS, n_chunks, _C)
    dst3 = dst.reshape(_NS, n_chunks, _C)

    xt1 = _tc_stage1(x, W1, b1)
    acc, deg = _sc_agg(xt1[:, :dh], xt1[:, dh:], src3, dst3, n, do_deg=True)
    xt2 = _tc_stage2(acc[0], acc[1], deg[0], deg[1], W2, b2)
    acc2, _ = _sc_agg(xt2[:, :dh], xt2[:, dh:], src3, dst3, n, do_deg=False)
    return _tc_stage3(acc2[0], acc2[1], deg[0], deg[1])
